# Initial kernel scaffold; baseline (speedup 1.0000x reference)
import jax
import jax.numpy as jnp
from jax.experimental import pallas as pl


def kernel(x, pos, batch, edge_index, W_in, b_in, W_rbf, W_msg, W_upd, W_emb, W_fc0, b_fc0, W_fc1, b_fc1, W_fc2, b_fc2, W_fc3, b_fc3):
    raise NotImplementedError("write your pallas kernel here")


# speedup vs baseline: 2.3338x; 2.3338x over previous
import functools

import jax
import jax.numpy as jnp
from jax import lax
from jax.experimental import pallas as pl
from jax.experimental.pallas import tpu as pltpu
from jax.experimental.pallas import tpu_sc as plsc

N = 10000
E = 320000
B = 100
D_IN = 128
D_NODE = 128
D_MSG = 64
D_EMB = 256
N_RADIAL = 32
CUTOFF = 6.0
FC_DEPTH = 256

NC = 2
NS = 16
NW = NC * NS
EPAD = 327680
EPT = EPAD // NW
CB = 128
NCB = EPT // CB
CE = 64
NCH = EPT // CE
RPT = N // NS

def _sc_mesh():
    return plsc.VectorSubcoreMesh(core_axis_name="c", subcore_axis_name="s")


_sc_params = pltpu.CompilerParams(needs_layout_passes=False)


def _node_embed_body(x_ref, win_ref, bin_ref, wmsg_ref, h_ref, hw_ref):
    h = jax.nn.gelu(
        jnp.dot(x_ref[...], win_ref[...], preferred_element_type=jnp.float32)
        + bin_ref[...]
    )
    h_ref[...] = h
    hw_ref[:, 0:D_MSG] = jnp.dot(h, wmsg_ref[...],
                                 preferred_element_type=jnp.float32)
    hw_ref[:, D_MSG:] = jnp.zeros_like(hw_ref[:, D_MSG:])


def _node_embed(x, w_in, b_in, w_msg):
    blk = 1000
    grid = N // blk
    return pl.pallas_call(
        _node_embed_body,
        grid=(grid,),
        in_specs=[
            pl.BlockSpec((blk, D_IN), lambda i: (i, 0)),
            pl.BlockSpec((D_IN, D_NODE), lambda i: (0, 0)),
            pl.BlockSpec((1, D_NODE), lambda i: (0, 0)),
            pl.BlockSpec((D_NODE, D_MSG), lambda i: (0, 0)),
        ],
        out_specs=[
            pl.BlockSpec((blk, D_NODE), lambda i: (i, 0)),
            pl.BlockSpec((blk, 2 * D_MSG), lambda i: (i, 0)),
        ],
        out_shape=[
            jax.ShapeDtypeStruct((N, D_NODE), jnp.float32),
            jax.ShapeDtypeStruct((N, 2 * D_MSG), jnp.float32),
        ],
    )(x, w_in, b_in, w_msg)


def _dist2_body(px_hbm, py_hbm, pz_hbm, src_hbm, dst_hbm, out_hbm,
                px_v, py_v, pz_v, src_v, dst_v, out_v):
    cid = lax.axis_index("c")
    sid = lax.axis_index("s")
    wid = sid * NC + cid
    base = wid * EPT
    pltpu.sync_copy(px_hbm, px_v)
    pltpu.sync_copy(py_hbm, py_v)
    pltpu.sync_copy(pz_hbm, pz_v)
    pltpu.sync_copy(src_hbm.at[pl.ds(wid * NCB, NCB)], src_v)
    pltpu.sync_copy(dst_hbm.at[pl.ds(wid * NCB, NCB)], dst_v)

    def body(r, carry):
        for cc in range(CB // 16):
            o = r * CB + cc * 16
            si = src_v[r, pl.ds(cc * 16, 16)]
            di = dst_v[r, pl.ds(cc * 16, 16)]
            dx = plsc.load_gather(px_v, [di]) - plsc.load_gather(px_v, [si])
            dy = plsc.load_gather(py_v, [di]) - plsc.load_gather(py_v, [si])
            dz = plsc.load_gather(pz_v, [di]) - plsc.load_gather(pz_v, [si])
            out_v[pl.ds(o, 16)] = dx * dx + dy * dy + dz * dz + 1e-8
        return carry

    lax.fori_loop(0, NCB, body, 0)
    pltpu.sync_copy(out_v, out_hbm.at[pl.ds(base, EPT)])


def _dist2_sc(px, py, pz, srcp, dstp):
    f = pl.kernel(
        _dist2_body,
        mesh=_sc_mesh(),
        compiler_params=_sc_params,
        out_type=jax.ShapeDtypeStruct((EPAD,), jnp.float32),
        scratch_types=[
            pltpu.VMEM((N,), jnp.float32),
            pltpu.VMEM((N,), jnp.float32),
            pltpu.VMEM((N,), jnp.float32),
            pltpu.VMEM((NCB, CB), jnp.int32),
            pltpu.VMEM((NCB, CB), jnp.int32),
            pltpu.VMEM((EPT,), jnp.float32),
        ],
    )
    return f(px, py, pz, srcp, dstp)


def _edge_w_body(d2_ref, wrbf_ref, ew_ref):
    i = pl.program_id(0)
    be = d2_ref.shape[0]
    d2 = d2_ref[...]
    d = jnp.sqrt(d2)
    step = CUTOFF / (N_RADIAL - 1)
    c = lax.broadcasted_iota(jnp.int32, (be, N_RADIAL), 1).astype(jnp.float32) * step
    width = CUTOFF / N_RADIAL
    rbf = jnp.exp(-((d - c) ** 2) * (1.0 / (2.0 * width * width)))
    env = 0.5 * (jnp.cos(jnp.pi * jnp.clip(d / CUTOFF, 0.0, 1.0)) + 1.0)
    gid = i * be + lax.broadcasted_iota(jnp.int32, (be, 1), 0)
    mask = (gid < E).astype(jnp.float32)
    ew_ref[...] = jnp.dot(rbf, wrbf_ref[...],
                          preferred_element_type=jnp.float32) * (env * mask)


def _edge_w(d2_col, w_rbf):
    be = 2048
    grid = EPAD // be
    return pl.pallas_call(
        _edge_w_body,
        grid=(grid,),
        in_specs=[
            pl.BlockSpec((be, 1), lambda i: (i, 0)),
            pl.BlockSpec((N_RADIAL, D_MSG), lambda i: (0, 0)),
        ],
        out_specs=pl.BlockSpec((be, D_MSG), lambda i: (i, 0)),
        out_shape=jax.ShapeDtypeStruct((EPAD, D_MSG), jnp.float32),
    )(d2_col, w_rbf)


def _scatter_body(hw_hbm, ew_hbm, src_hbm, dst_hbm, out_hbm,
                  agg_sh, is0, is1, is2, id0, id1, id2,
                  rows_a, rows_b, ew_a, ew_b,
                  sem_a, sem_b, isem_a, isem_b, isem_c):
    cid = lax.axis_index("c")
    sid = lax.axis_index("s")
    wid = sid * NC + cid
    base = wid * EPT

    rows = (rows_a, rows_b)
    ews = (ew_a, ew_b)
    sems = (sem_a, sem_b)
    isrc = (is0, is1, is2)
    idst = (id0, id1, id2)
    isems = (isem_a, isem_b, isem_c)

    def zrow(r, carry):
        for cc in range(2 * D_MSG // 16):
            rows_a[r, pl.ds(cc * 16, 16)] = jnp.zeros((16,), jnp.float32)
        return carry

    lax.fori_loop(0, CE, zrow, 0)
    for k in range(9):
        pltpu.sync_copy(rows_a, agg_sh.at[pl.ds(sid * RPT + k * CE, CE)])
    pltpu.sync_copy(rows_a.at[pl.ds(0, RPT - 9 * CE)],
                    agg_sh.at[pl.ds(sid * RPT + 9 * CE, RPT - 9 * CE)])
    plsc.subcore_barrier()

    def iload(c, ib):
        eb = base + c * CE
        pltpu.async_copy(src_hbm.at[pl.ds(eb, CE)], isrc[ib], isems[ib])
        pltpu.async_copy(dst_hbm.at[pl.ds(eb, CE)], idst[ib], isems[ib])

    def iwait(c, ib):
        eb = base + c * CE
        pltpu.make_async_copy(src_hbm.at[pl.ds(eb, CE)], isrc[ib],
                              isems[ib]).wait()
        pltpu.make_async_copy(dst_hbm.at[pl.ds(eb, CE)], idst[ib],
                              isems[ib]).wait()

    def start(c, buf, ib):
        eb = base + c * CE
        pltpu.async_copy(hw_hbm.at[isrc[ib]], rows[buf], sems[buf])
        pltpu.async_copy(ew_hbm.at[pl.ds(eb, CE)], ews[buf], sems[buf])

    def finish(c, buf, ib):
        pltpu.make_async_copy(hw_hbm.at[isrc[ib]], rows[buf],
                              sems[buf]).wait()
        pltpu.make_async_copy(ew_hbm.at[pl.ds(base + c * CE, CE)],
                              ews[buf], sems[buf]).wait()

        def mrow(r, carry):
            for cc in range(D_MSG // 16):
                s = pl.ds(cc * 16, 16)
                rows[buf][r, s] = rows[buf][r, s] * ews[buf][r, s]
            return carry

        lax.fori_loop(0, CE, mrow, 0)
        pltpu.sync_copy(rows[buf], agg_sh.at[idst[ib]], add=True)

    iload(0, 0)
    iload(1, 1)
    iwait(0, 0)
    start(0, 0, 0)
    for c in range(1, NCH):
        iwait(c, c % 3)
        start(c, c % 2, c % 3)
        if c + 1 < NCH:
            iload(c + 1, (c + 1) % 3)
        finish(c - 1, (c - 1) % 2, (c - 1) % 3)
    finish(NCH - 1, (NCH - 1) % 2, (NCH - 1) % 3)

    plsc.subcore_barrier()
    pltpu.sync_copy(agg_sh.at[pl.ds(sid * RPT, RPT)], out_hbm.at[cid, sid])


def _scatter_sc(hw, ew, src1, dst1):
    f = pl.kernel(
        _scatter_body,
        mesh=_sc_mesh(),
        compiler_params=_sc_params,
        out_type=jax.ShapeDtypeStruct((NC, NS, RPT, 2 * D_MSG), jnp.float32),
        scratch_types=[
            pltpu.VMEM_SHARED((N, 2 * D_MSG), jnp.float32),
            pltpu.VMEM((CE,), jnp.int32),
            pltpu.VMEM((CE,), jnp.int32),
            pltpu.VMEM((CE,), jnp.int32),
            pltpu.VMEM((CE,), jnp.int32),
            pltpu.VMEM((CE,), jnp.int32),
            pltpu.VMEM((CE,), jnp.int32),
            pltpu.VMEM((CE, 2 * D_MSG), jnp.float32),
            pltpu.VMEM((CE, 2 * D_MSG), jnp.float32),
            pltpu.VMEM((CE, D_MSG), jnp.float32),
            pltpu.VMEM((CE, D_MSG), jnp.float32),
            pltpu.SemaphoreType.DMA,
            pltpu.SemaphoreType.DMA,
            pltpu.SemaphoreType.DMA,
            pltpu.SemaphoreType.DMA,
            pltpu.SemaphoreType.DMA,
        ],
    )
    return f(hw, ew, src1, dst1)


def _tail_body(agg_ref, h_ref, batch_ref, wupd_ref, wemb_ref,
               w0_ref, b0_ref, w1_ref, b1_ref, w2_ref, b2_ref, w3_ref, b3_ref,
               y_ref, acc_ref):
    i = pl.program_id(0)
    nblk = pl.num_programs(0)
    agg = (agg_ref[0] + agg_ref[1])[:, 0:D_MSG]
    h2 = h_ref[...] + jax.nn.gelu(
        jnp.dot(agg, wupd_ref[...], preferred_element_type=jnp.float32))
    g = jnp.dot(h2, wemb_ref[...], preferred_element_type=jnp.float32)
    brow = batch_ref[0]
    oht = (brow == lax.broadcasted_iota(jnp.int32, (B, brow.shape[1]), 0))
    contrib = jnp.dot(oht.astype(jnp.float32), g,
                      preferred_element_type=jnp.float32)

    @pl.when(i == 0)
    def _():
        acc_ref[...] = jnp.zeros_like(acc_ref)

    acc_ref[...] += contrib

    @pl.when(i == nblk - 1)
    def _():
        y = acc_ref[...]
        for w_ref, b_ref in ((w0_ref, b0_ref), (w1_ref, b1_ref),
                             (w2_ref, b2_ref), (w3_ref, b3_ref)):
            y = jax.nn.gelu(
                jnp.dot(y, w_ref[...], preferred_element_type=jnp.float32)
                + b_ref[...])
        y_ref[...] = y


def _tail(agg2, h, batch3, w_upd, w_emb, w0, b0, w1, b1, w2, b2, w3, b3):
    blk = 1000
    grid = N // blk
    fcspec = pl.BlockSpec((FC_DEPTH, FC_DEPTH), lambda i: (0, 0))
    bspec = pl.BlockSpec((1, FC_DEPTH), lambda i: (0, 0))
    return pl.pallas_call(
        _tail_body,
        grid=(grid,),
        in_specs=[
            pl.BlockSpec((NC, blk, 2 * D_MSG), lambda i: (0, i, 0)),
            pl.BlockSpec((blk, D_NODE), lambda i: (i, 0)),
            pl.BlockSpec((1, 1, blk), lambda i: (i, 0, 0)),
            pl.BlockSpec((D_MSG, D_NODE), lambda i: (0, 0)),
            pl.BlockSpec((D_NODE, D_EMB), lambda i: (0, 0)),
            pl.BlockSpec((D_EMB, FC_DEPTH), lambda i: (0, 0)),
            bspec, fcspec, bspec, fcspec, bspec, fcspec, bspec,
        ],
        out_specs=pl.BlockSpec((B, FC_DEPTH), lambda i: (0, 0)),
        out_shape=jax.ShapeDtypeStruct((B, FC_DEPTH), jnp.float32),
        scratch_shapes=[pltpu.VMEM((B, FC_DEPTH), jnp.float32)],
    )(agg2, h, batch3, w_upd, w_emb, w0, b0, w1, b1, w2, b2, w3, b3)


def kernel(x, pos, batch, edge_index, W_in, b_in, W_rbf, W_msg, W_upd, W_emb,
           W_fc0, b_fc0, W_fc1, b_fc1, W_fc2, b_fc2, W_fc3, b_fc3):
    src = edge_index[0].astype(jnp.int32)
    dst = edge_index[1].astype(jnp.int32)
    padz = jnp.zeros((EPAD - E,), jnp.int32)
    src1 = jnp.concatenate([src, padz])
    dst1 = jnp.concatenate([dst, padz])
    srcp = src1.reshape(EPAD // CB, CB)
    dstp = dst1.reshape(EPAD // CB, CB)
    px = pos[:, 0]
    py = pos[:, 1]
    pz = pos[:, 2]

    h, hw = _node_embed(x, W_in, b_in.reshape(1, D_NODE), W_msg)
    d2 = _dist2_sc(px, py, pz, srcp, dstp)
    ew = _edge_w(d2.reshape(EPAD, 1), W_rbf)
    agg2 = _scatter_sc(hw, ew, src1, dst1).reshape(NC, N, 2 * D_MSG)
    batch3 = batch.astype(jnp.int32).reshape(N // 1000, 1, 1000)
    y = _tail(agg2, h, batch3, W_upd, W_emb,
              W_fc0, b_fc0.reshape(1, FC_DEPTH), W_fc1, b_fc1.reshape(1, FC_DEPTH),
              W_fc2, b_fc2.reshape(1, FC_DEPTH), W_fc3, b_fc3.reshape(1, FC_DEPTH))
    return y


# speedup vs baseline: 2.3490x; 1.0065x over previous
import functools

import jax
import jax.numpy as jnp
from jax import lax
from jax.experimental import pallas as pl
from jax.experimental.pallas import tpu as pltpu
from jax.experimental.pallas import tpu_sc as plsc

N = 10000
E = 320000
B = 100
D_IN = 128
D_NODE = 128
D_MSG = 64
D_EMB = 256
N_RADIAL = 32
CUTOFF = 6.0
FC_DEPTH = 256

NC = 2
NS = 16
NW = NC * NS
EPAD = 327680
EPT = EPAD // NW
CB = 128
NCB = EPT // CB
CE = 64
NCH = EPT // CE
RPT = N // NS

def _sc_mesh():
    return plsc.VectorSubcoreMesh(core_axis_name="c", subcore_axis_name="s")


_sc_params = pltpu.CompilerParams(needs_layout_passes=False)


def _node_embed_body(x_ref, win_ref, bin_ref, wmsg_ref, h_ref, hw_ref):
    h = jax.nn.gelu(
        jnp.dot(x_ref[...], win_ref[...], preferred_element_type=jnp.float32)
        + bin_ref[...]
    )
    h_ref[...] = h
    hw_ref[:, 0:D_MSG] = jnp.dot(h, wmsg_ref[...],
                                 preferred_element_type=jnp.float32)
    hw_ref[:, D_MSG:] = jnp.zeros_like(hw_ref[:, D_MSG:])


def _node_embed(x, w_in, b_in, w_msg):
    blk = 1000
    grid = N // blk
    return pl.pallas_call(
        _node_embed_body,
        grid=(grid,),
        in_specs=[
            pl.BlockSpec((blk, D_IN), lambda i: (i, 0)),
            pl.BlockSpec((D_IN, D_NODE), lambda i: (0, 0)),
            pl.BlockSpec((1, D_NODE), lambda i: (0, 0)),
            pl.BlockSpec((D_NODE, D_MSG), lambda i: (0, 0)),
        ],
        out_specs=[
            pl.BlockSpec((blk, D_NODE), lambda i: (i, 0)),
            pl.BlockSpec((blk, 2 * D_MSG), lambda i: (i, 0)),
        ],
        out_shape=[
            jax.ShapeDtypeStruct((N, D_NODE), jnp.float32),
            jax.ShapeDtypeStruct((N, 2 * D_MSG), jnp.float32),
        ],
    )(x, w_in, b_in, w_msg)


def _dist2_body(px_hbm, py_hbm, pz_hbm, src_hbm, dst_hbm, out_hbm,
                px_v, py_v, pz_v, src_v, dst_v, out_v):
    cid = lax.axis_index("c")
    sid = lax.axis_index("s")
    wid = sid * NC + cid
    base = wid * EPT
    pltpu.sync_copy(px_hbm, px_v)
    pltpu.sync_copy(py_hbm, py_v)
    pltpu.sync_copy(pz_hbm, pz_v)
    pltpu.sync_copy(src_hbm.at[pl.ds(wid * NCB, NCB)], src_v)
    pltpu.sync_copy(dst_hbm.at[pl.ds(wid * NCB, NCB)], dst_v)

    def body(r, carry):
        for cc in range(CB // 16):
            o = r * CB + cc * 16
            si = src_v[r, pl.ds(cc * 16, 16)]
            di = dst_v[r, pl.ds(cc * 16, 16)]
            dx = plsc.load_gather(px_v, [di]) - plsc.load_gather(px_v, [si])
            dy = plsc.load_gather(py_v, [di]) - plsc.load_gather(py_v, [si])
            dz = plsc.load_gather(pz_v, [di]) - plsc.load_gather(pz_v, [si])
            out_v[pl.ds(o, 16)] = dx * dx + dy * dy + dz * dz + 1e-8
        return carry

    lax.fori_loop(0, NCB, body, 0)
    pltpu.sync_copy(out_v, out_hbm.at[pl.ds(base, EPT)])


def _dist2_sc(px, py, pz, srcp, dstp):
    f = pl.kernel(
        _dist2_body,
        mesh=_sc_mesh(),
        compiler_params=_sc_params,
        out_type=jax.ShapeDtypeStruct((EPAD,), jnp.float32),
        scratch_types=[
            pltpu.VMEM((N,), jnp.float32),
            pltpu.VMEM((N,), jnp.float32),
            pltpu.VMEM((N,), jnp.float32),
            pltpu.VMEM((NCB, CB), jnp.int32),
            pltpu.VMEM((NCB, CB), jnp.int32),
            pltpu.VMEM((EPT,), jnp.float32),
        ],
    )
    return f(px, py, pz, srcp, dstp)


def _edge_w_body(d2_ref, wrbf_ref, ew_ref):
    i = pl.program_id(0)
    be = d2_ref.shape[0]
    d2 = d2_ref[...]
    d = jnp.sqrt(d2)
    step = CUTOFF / (N_RADIAL - 1)
    c = lax.broadcasted_iota(jnp.int32, (be, N_RADIAL), 1).astype(jnp.float32) * step
    width = CUTOFF / N_RADIAL
    rbf = jnp.exp(-((d - c) ** 2) * (1.0 / (2.0 * width * width)))
    env = 0.5 * (jnp.cos(jnp.pi * jnp.clip(d / CUTOFF, 0.0, 1.0)) + 1.0)
    gid = i * be + lax.broadcasted_iota(jnp.int32, (be, 1), 0)
    mask = (gid < E).astype(jnp.float32)
    ew_ref[...] = jnp.dot(rbf, wrbf_ref[...],
                          preferred_element_type=jnp.float32) * (env * mask)


def _edge_w(d2_col, w_rbf):
    be = 2048
    grid = EPAD // be
    return pl.pallas_call(
        _edge_w_body,
        grid=(grid,),
        in_specs=[
            pl.BlockSpec((be, 1), lambda i: (i, 0)),
            pl.BlockSpec((N_RADIAL, D_MSG), lambda i: (0, 0)),
        ],
        out_specs=pl.BlockSpec((be, D_MSG), lambda i: (i, 0)),
        out_shape=jax.ShapeDtypeStruct((EPAD, D_MSG), jnp.float32),
    )(d2_col, w_rbf)


def _scatter_body(hw_hbm, ew_hbm, src_hbm, dst_hbm, out_hbm,
                  agg_sh,
                  is0, is1, is2, is3, is4, is5, is6, is7,
                  id0, id1, id2, id3, id4, id5, id6, id7,
                  rows0, rows1, ew0, ew1,
                  g0, g1, s0, s1,
                  i0, i1, i2, i3, i4, i5, i6, i7):
    cid = lax.axis_index("c")
    sid = lax.axis_index("s")
    wid = sid * NC + cid
    base = wid * EPT

    rows = (rows0, rows1)
    ews = (ew0, ew1)
    gsem = (g0, g1)
    ssem = (s0, s1)
    isrc = (is0, is1, is2, is3, is4, is5, is6, is7)
    idst = (id0, id1, id2, id3, id4, id5, id6, id7)
    isem = (i0, i1, i2, i3, i4, i5, i6, i7)

    def zrow(r, carry):
        for cc in range(2 * D_MSG // 16):
            rows0[r, pl.ds(cc * 16, 16)] = jnp.zeros((16,), jnp.float32)
        return carry

    lax.fori_loop(0, CE, zrow, 0)
    nz = RPT // CE
    for k in range(nz):
        pltpu.sync_copy(rows0, agg_sh.at[pl.ds(sid * RPT + k * CE, CE)])
    pltpu.sync_copy(rows0.at[pl.ds(0, RPT - nz * CE)],
                    agg_sh.at[pl.ds(sid * RPT + nz * CE, RPT - nz * CE)])
    plsc.subcore_barrier()

    def iload(c, s8):
        eb = base + c * CE
        pltpu.async_copy(src_hbm.at[pl.ds(eb, CE)], isrc[s8], isem[s8])
        pltpu.async_copy(dst_hbm.at[pl.ds(eb, CE)], idst[s8], isem[s8])

    def iwait(c, s8):
        eb = base + c * CE
        pltpu.make_async_copy(src_hbm.at[pl.ds(eb, CE)], isrc[s8],
                              isem[s8]).wait()
        pltpu.make_async_copy(dst_hbm.at[pl.ds(eb, CE)], idst[s8],
                              isem[s8]).wait()

    def start(c, s2, s8):
        eb = base + c * CE
        pltpu.async_copy(hw_hbm.at[isrc[s8]], rows[s2], gsem[s2])
        pltpu.async_copy(ew_hbm.at[pl.ds(eb, CE)], ews[s2], gsem[s2])

    def swait(s2, s8):
        pltpu.make_async_copy(rows[s2], agg_sh.at[idst[s8]],
                              ssem[s2]).wait()

    def finish(c, s2, s8):
        pltpu.make_async_copy(hw_hbm.at[isrc[s8]], rows[s2],
                              gsem[s2]).wait()
        pltpu.make_async_copy(ew_hbm.at[pl.ds(base + c * CE, CE)],
                              ews[s2], gsem[s2]).wait()

        def mrow(r, carry):
            for u in range(4):
                rr = r * 4 + u
                for cc in range(D_MSG // 16):
                    s = pl.ds(cc * 16, 16)
                    rows[s2][rr, s] = rows[s2][rr, s] * ews[s2][rr, s]
            return carry

        lax.fori_loop(0, CE // 4, mrow, 0)
        pltpu.async_copy(rows[s2], agg_sh.at[idst[s8]], ssem[s2], add=True)

    R = 8
    iload(0, 0)
    iload(1, 1)
    iwait(0, 0)
    start(0, 0, 0)
    iload(2, 2)
    for c in range(1, R):
        iwait(c, c % 8)
        if c >= 2:
            swait((c - 2) % 2, (c - 2) % 8)
        start(c, c % 2, c % 8)
        iload(c + 2, (c + 2) % 8)
        finish(c - 1, (c - 1) % 2, (c - 1) % 8)

    def group(gi, carry):
        c0 = R + gi * R
        for b in range(R):
            c = c0 + b
            iwait(c, b % 8)
            swait(b % 2, (b - 2) % 8)
            start(c, b % 2, b % 8)
            iload(c + 2, (b + 2) % 8)
            finish(c - 1, (b - 1) % 2, (b - 1) % 8)
        return carry

    lax.fori_loop(0, (NCH - 2 * R) // R, group, 0)

    for c in range(NCH - R, NCH):
        iwait(c, c % 8)
        swait((c - 2) % 2, (c - 2) % 8)
        start(c, c % 2, c % 8)
        if c + 2 < NCH:
            iload(c + 2, (c + 2) % 8)
        finish(c - 1, (c - 1) % 2, (c - 1) % 8)
    finish(NCH - 1, (NCH - 1) % 2, (NCH - 1) % 8)
    swait((NCH - 2) % 2, (NCH - 2) % 8)
    swait((NCH - 1) % 2, (NCH - 1) % 8)

    plsc.subcore_barrier()
    pltpu.sync_copy(agg_sh.at[pl.ds(sid * RPT, RPT)], out_hbm.at[cid, sid])


def _scatter_sc(hw, ew, src1, dst1):
    f = pl.kernel(
        _scatter_body,
        mesh=_sc_mesh(),
        compiler_params=_sc_params,
        out_type=jax.ShapeDtypeStruct((NC, NS, RPT, 2 * D_MSG), jnp.float32),
        scratch_types=(
            [pltpu.VMEM_SHARED((N, 2 * D_MSG), jnp.float32)]
            + [pltpu.VMEM((CE,), jnp.int32) for _ in range(16)]
            + [pltpu.VMEM((CE, 2 * D_MSG), jnp.float32) for _ in range(2)]
            + [pltpu.VMEM((CE, D_MSG), jnp.float32) for _ in range(2)]
            + [pltpu.SemaphoreType.DMA for _ in range(12)]
        ),
    )
    return f(hw, ew, src1, dst1)


def _tail_body(agg_ref, h_ref, batch_ref, wupd_ref, wemb_ref,
               w0_ref, b0_ref, w1_ref, b1_ref, w2_ref, b2_ref, w3_ref, b3_ref,
               y_ref, acc_ref):
    i = pl.program_id(0)
    nblk = pl.num_programs(0)
    agg = (agg_ref[0] + agg_ref[1])[:, 0:D_MSG]
    h2 = h_ref[...] + jax.nn.gelu(
        jnp.dot(agg, wupd_ref[...], preferred_element_type=jnp.float32))
    g = jnp.dot(h2, wemb_ref[...], preferred_element_type=jnp.float32)
    brow = batch_ref[0]
    oht = (brow == lax.broadcasted_iota(jnp.int32, (B, brow.shape[1]), 0))
    contrib = jnp.dot(oht.astype(jnp.float32), g,
                      preferred_element_type=jnp.float32)

    @pl.when(i == 0)
    def _():
        acc_ref[...] = jnp.zeros_like(acc_ref)

    acc_ref[...] += contrib

    @pl.when(i == nblk - 1)
    def _():
        y = acc_ref[...]
        for w_ref, b_ref in ((w0_ref, b0_ref), (w1_ref, b1_ref),
                             (w2_ref, b2_ref), (w3_ref, b3_ref)):
            y = jax.nn.gelu(
                jnp.dot(y, w_ref[...], preferred_element_type=jnp.float32)
                + b_ref[...])
        y_ref[...] = y


def _tail(agg2, h, batch3, w_upd, w_emb, w0, b0, w1, b1, w2, b2, w3, b3):
    blk = 1000
    grid = N // blk
    fcspec = pl.BlockSpec((FC_DEPTH, FC_DEPTH), lambda i: (0, 0))
    bspec = pl.BlockSpec((1, FC_DEPTH), lambda i: (0, 0))
    return pl.pallas_call(
        _tail_body,
        grid=(grid,),
        in_specs=[
            pl.BlockSpec((NC, blk, 2 * D_MSG), lambda i: (0, i, 0)),
            pl.BlockSpec((blk, D_NODE), lambda i: (i, 0)),
            pl.BlockSpec((1, 1, blk), lambda i: (i, 0, 0)),
            pl.BlockSpec((D_MSG, D_NODE), lambda i: (0, 0)),
            pl.BlockSpec((D_NODE, D_EMB), lambda i: (0, 0)),
            pl.BlockSpec((D_EMB, FC_DEPTH), lambda i: (0, 0)),
            bspec, fcspec, bspec, fcspec, bspec, fcspec, bspec,
        ],
        out_specs=pl.BlockSpec((B, FC_DEPTH), lambda i: (0, 0)),
        out_shape=jax.ShapeDtypeStruct((B, FC_DEPTH), jnp.float32),
        scratch_shapes=[pltpu.VMEM((B, FC_DEPTH), jnp.float32)],
    )(agg2, h, batch3, w_upd, w_emb, w0, b0, w1, b1, w2, b2, w3, b3)


def kernel(x, pos, batch, edge_index, W_in, b_in, W_rbf, W_msg, W_upd, W_emb,
           W_fc0, b_fc0, W_fc1, b_fc1, W_fc2, b_fc2, W_fc3, b_fc3):
    src = edge_index[0].astype(jnp.int32)
    dst = edge_index[1].astype(jnp.int32)
    padz = jnp.zeros((EPAD - E,), jnp.int32)
    src1 = jnp.concatenate([src, padz])
    dst1 = jnp.concatenate([dst, padz])
    srcp = src1.reshape(EPAD // CB, CB)
    dstp = dst1.reshape(EPAD // CB, CB)
    px = pos[:, 0]
    py = pos[:, 1]
    pz = pos[:, 2]

    h, hw = _node_embed(x, W_in, b_in.reshape(1, D_NODE), W_msg)
    d2 = _dist2_sc(px, py, pz, srcp, dstp)
    ew = _edge_w(d2.reshape(EPAD, 1), W_rbf)
    agg2 = _scatter_sc(hw, ew, src1, dst1).reshape(NC, N, 2 * D_MSG)
    batch3 = batch.astype(jnp.int32).reshape(N // 1000, 1, 1000)
    y = _tail(agg2, h, batch3, W_upd, W_emb,
              W_fc0, b_fc0.reshape(1, FC_DEPTH), W_fc1, b_fc1.reshape(1, FC_DEPTH),
              W_fc2, b_fc2.reshape(1, FC_DEPTH), W_fc3, b_fc3.reshape(1, FC_DEPTH))
    return y


# speedup vs baseline: 5.2307x; 2.2268x over previous
import functools

import jax
import jax.numpy as jnp
from jax import lax
from jax.experimental import pallas as pl
from jax.experimental.pallas import tpu as pltpu
from jax.experimental.pallas import tpu_sc as plsc

N = 10000
E = 320000
B = 100
D_IN = 128
D_NODE = 128
D_MSG = 64
D_EMB = 256
N_RADIAL = 32
CUTOFF = 6.0
FC_DEPTH = 256

NC = 2
NS = 16
NW = NC * NS
EPAD = 327680
EPT = EPAD // NW
CB = 128
NCB = EPT // CB
CE = 64
NCH = EPT // CE
RPT = N // NS

def _sc_mesh():
    return plsc.VectorSubcoreMesh(core_axis_name="c", subcore_axis_name="s")


_sc_params = pltpu.CompilerParams(needs_layout_passes=False)


def _node_embed_body(x_ref, win_ref, bin_ref, wmsg_ref, h_ref, hw_ref):
    h = jax.nn.gelu(
        jnp.dot(x_ref[...], win_ref[...], preferred_element_type=jnp.float32)
        + bin_ref[...]
    )
    h_ref[...] = h
    hw_ref[:, 0:D_MSG] = jnp.dot(h, wmsg_ref[...],
                                 preferred_element_type=jnp.float32)
    hw_ref[:, D_MSG:] = jnp.zeros_like(hw_ref[:, D_MSG:])


def _node_embed(x, w_in, b_in, w_msg):
    blk = 1000
    grid = N // blk
    return pl.pallas_call(
        _node_embed_body,
        grid=(grid,),
        in_specs=[
            pl.BlockSpec((blk, D_IN), lambda i: (i, 0)),
            pl.BlockSpec((D_IN, D_NODE), lambda i: (0, 0)),
            pl.BlockSpec((1, D_NODE), lambda i: (0, 0)),
            pl.BlockSpec((D_NODE, D_MSG), lambda i: (0, 0)),
        ],
        out_specs=[
            pl.BlockSpec((blk, D_NODE), lambda i: (i, 0)),
            pl.BlockSpec((blk, 2 * D_MSG), lambda i: (i, 0)),
        ],
        out_shape=[
            jax.ShapeDtypeStruct((N, D_NODE), jnp.float32),
            jax.ShapeDtypeStruct((N, 2 * D_MSG), jnp.float32),
        ],
    )(x, w_in, b_in, w_msg)


def _dist2_body(px_hbm, py_hbm, pz_hbm, src_hbm, dst_hbm, out_hbm,
                px_v, py_v, pz_v, src_v, dst_v, out_v):
    cid = lax.axis_index("c")
    sid = lax.axis_index("s")
    wid = sid * NC + cid
    base = wid * EPT
    pltpu.sync_copy(px_hbm, px_v)
    pltpu.sync_copy(py_hbm, py_v)
    pltpu.sync_copy(pz_hbm, pz_v)
    pltpu.sync_copy(src_hbm.at[pl.ds(wid * NCB, NCB)], src_v)
    pltpu.sync_copy(dst_hbm.at[pl.ds(wid * NCB, NCB)], dst_v)

    def body(r, carry):
        for cc in range(CB // 16):
            o = r * CB + cc * 16
            si = src_v[r, pl.ds(cc * 16, 16)]
            di = dst_v[r, pl.ds(cc * 16, 16)]
            dx = plsc.load_gather(px_v, [di]) - plsc.load_gather(px_v, [si])
            dy = plsc.load_gather(py_v, [di]) - plsc.load_gather(py_v, [si])
            dz = plsc.load_gather(pz_v, [di]) - plsc.load_gather(pz_v, [si])
            out_v[pl.ds(o, 16)] = dx * dx + dy * dy + dz * dz + 1e-8
        return carry

    lax.fori_loop(0, NCB, body, 0)
    pltpu.sync_copy(out_v, out_hbm.at[pl.ds(base, EPT)])


def _dist2_sc(px, py, pz, srcp, dstp):
    f = pl.kernel(
        _dist2_body,
        mesh=_sc_mesh(),
        compiler_params=_sc_params,
        out_type=jax.ShapeDtypeStruct((EPAD,), jnp.float32),
        scratch_types=[
            pltpu.VMEM((N,), jnp.float32),
            pltpu.VMEM((N,), jnp.float32),
            pltpu.VMEM((N,), jnp.float32),
            pltpu.VMEM((NCB, CB), jnp.int32),
            pltpu.VMEM((NCB, CB), jnp.int32),
            pltpu.VMEM((EPT,), jnp.float32),
        ],
    )
    return f(px, py, pz, srcp, dstp)


def _edge_w_body(d2_ref, s_ref, w4_ref, ew_ref):
    i = pl.program_id(0)
    rows = d2_ref.shape[0]
    dw = jnp.sqrt(d2_ref[...])
    gid = ((i * rows + lax.broadcasted_iota(jnp.int32, (rows, 128), 0)) * 128
           + lax.broadcasted_iota(jnp.int32, (rows, 128), 1))
    maskw = (gid < E).astype(jnp.float32)
    envw = (0.5 * (jnp.cos(jnp.pi * jnp.clip(dw * (1.0 / CUTOFF), 0.0, 1.0))
                   + 1.0)) * maskw
    dt = dw.T
    et = envw.T
    step = CUTOFF / (N_RADIAL - 1)
    c4 = (lax.broadcasted_iota(jnp.int32, (1, 128), 1) % N_RADIAL
          ).astype(jnp.float32) * step
    width = CUTOFF / N_RADIAL
    gamma = 1.0 / (2.0 * width * width)
    s_mat = s_ref[...]
    for g in range(rows // 4):
        d4 = jnp.dot(dt[:, 4 * g:4 * g + 4], s_mat,
                     preferred_element_type=jnp.float32)
        e4 = jnp.dot(et[:, 4 * g:4 * g + 4], s_mat,
                     preferred_element_type=jnp.float32)
        r4 = jnp.exp(-((d4 - c4) ** 2) * gamma) * e4
        ew4 = jnp.dot(r4, w4_ref[...], preferred_element_type=jnp.float32)
        for b in range(4):
            ew_ref[pl.ds((4 * g + b) * 128, 128), :] = (
                ew4[:, b * D_MSG:(b + 1) * D_MSG])


def _edge_w(d2_2d, w_rbf):
    rows = 64
    grid = (EPAD // 128) // rows
    s_mat = jnp.repeat(jnp.eye(4, dtype=jnp.float32), N_RADIAL, axis=1)
    w4 = jnp.kron(jnp.eye(4, dtype=jnp.float32), w_rbf)
    return pl.pallas_call(
        _edge_w_body,
        grid=(grid,),
        in_specs=[
            pl.BlockSpec((rows, 128), lambda i: (i, 0)),
            pl.BlockSpec((4, 128), lambda i: (0, 0)),
            pl.BlockSpec((4 * N_RADIAL, 4 * D_MSG), lambda i: (0, 0)),
        ],
        out_specs=pl.BlockSpec((rows * 128, D_MSG), lambda i: (i, 0)),
        out_shape=jax.ShapeDtypeStruct((EPAD, D_MSG), jnp.float32),
    )(d2_2d, s_mat, w4)


def _scatter_body(hw_hbm, ew_hbm, src_hbm, dst_hbm, out_hbm,
                  agg_sh,
                  is0, is1, is2, is3, is4, is5, is6, is7,
                  id0, id1, id2, id3, id4, id5, id6, id7,
                  rows0, rows1, ew0, ew1,
                  g0, g1, s0, s1,
                  i0, i1, i2, i3, i4, i5, i6, i7):
    cid = lax.axis_index("c")
    sid = lax.axis_index("s")
    wid = sid * NC + cid
    base = wid * EPT

    rows = (rows0, rows1)
    ews = (ew0, ew1)
    gsem = (g0, g1)
    ssem = (s0, s1)
    isrc = (is0, is1, is2, is3, is4, is5, is6, is7)
    idst = (id0, id1, id2, id3, id4, id5, id6, id7)
    isem = (i0, i1, i2, i3, i4, i5, i6, i7)

    def zrow(r, carry):
        for cc in range(2 * D_MSG // 16):
            rows0[r, pl.ds(cc * 16, 16)] = jnp.zeros((16,), jnp.float32)
        return carry

    lax.fori_loop(0, CE, zrow, 0)
    nz = RPT // CE
    for k in range(nz):
        pltpu.sync_copy(rows0, agg_sh.at[pl.ds(sid * RPT + k * CE, CE)])
    pltpu.sync_copy(rows0.at[pl.ds(0, RPT - nz * CE)],
                    agg_sh.at[pl.ds(sid * RPT + nz * CE, RPT - nz * CE)])
    plsc.subcore_barrier()

    def iload(c, s8):
        eb = base + c * CE
        pltpu.async_copy(src_hbm.at[pl.ds(eb, CE)], isrc[s8], isem[s8])
        pltpu.async_copy(dst_hbm.at[pl.ds(eb, CE)], idst[s8], isem[s8])

    def iwait(c, s8):
        eb = base + c * CE
        pltpu.make_async_copy(src_hbm.at[pl.ds(eb, CE)], isrc[s8],
                              isem[s8]).wait()
        pltpu.make_async_copy(dst_hbm.at[pl.ds(eb, CE)], idst[s8],
                              isem[s8]).wait()

    def start(c, s2, s8):
        eb = base + c * CE
        pltpu.async_copy(hw_hbm.at[isrc[s8]], rows[s2], gsem[s2])
        pltpu.async_copy(ew_hbm.at[pl.ds(eb, CE)], ews[s2], gsem[s2])

    def swait(s2, s8):
        pltpu.make_async_copy(rows[s2], agg_sh.at[idst[s8]],
                              ssem[s2]).wait()

    def finish(c, s2, s8):
        pltpu.make_async_copy(hw_hbm.at[isrc[s8]], rows[s2],
                              gsem[s2]).wait()
        pltpu.make_async_copy(ew_hbm.at[pl.ds(base + c * CE, CE)],
                              ews[s2], gsem[s2]).wait()

        def mrow(r, carry):
            for u in range(4):
                rr = r * 4 + u
                for cc in range(D_MSG // 16):
                    s = pl.ds(cc * 16, 16)
                    rows[s2][rr, s] = rows[s2][rr, s] * ews[s2][rr, s]
            return carry

        lax.fori_loop(0, CE // 4, mrow, 0)
        pltpu.async_copy(rows[s2], agg_sh.at[idst[s8]], ssem[s2], add=True)

    R = 8
    iload(0, 0)
    iload(1, 1)
    iwait(0, 0)
    start(0, 0, 0)
    iload(2, 2)
    for c in range(1, R):
        iwait(c, c % 8)
        if c >= 2:
            swait((c - 2) % 2, (c - 2) % 8)
        start(c, c % 2, c % 8)
        iload(c + 2, (c + 2) % 8)
        finish(c - 1, (c - 1) % 2, (c - 1) % 8)

    def group(gi, carry):
        c0 = R + gi * R
        for b in range(R):
            c = c0 + b
            iwait(c, b % 8)
            swait(b % 2, (b - 2) % 8)
            start(c, b % 2, b % 8)
            iload(c + 2, (b + 2) % 8)
            finish(c - 1, (b - 1) % 2, (b - 1) % 8)
        return carry

    lax.fori_loop(0, (NCH - 2 * R) // R, group, 0)

    for c in range(NCH - R, NCH):
        iwait(c, c % 8)
        swait((c - 2) % 2, (c - 2) % 8)
        start(c, c % 2, c % 8)
        if c + 2 < NCH:
            iload(c + 2, (c + 2) % 8)
        finish(c - 1, (c - 1) % 2, (c - 1) % 8)
    finish(NCH - 1, (NCH - 1) % 2, (NCH - 1) % 8)
    swait((NCH - 2) % 2, (NCH - 2) % 8)
    swait((NCH - 1) % 2, (NCH - 1) % 8)

    plsc.subcore_barrier()
    pltpu.sync_copy(agg_sh.at[pl.ds(sid * RPT, RPT)], out_hbm.at[cid, sid])


def _scatter_sc(hw, ew, src1, dst1):
    f = pl.kernel(
        _scatter_body,
        mesh=_sc_mesh(),
        compiler_params=_sc_params,
        out_type=jax.ShapeDtypeStruct((NC, NS, RPT, 2 * D_MSG), jnp.float32),
        scratch_types=(
            [pltpu.VMEM_SHARED((N, 2 * D_MSG), jnp.float32)]
            + [pltpu.VMEM((CE,), jnp.int32) for _ in range(16)]
            + [pltpu.VMEM((CE, 2 * D_MSG), jnp.float32) for _ in range(2)]
            + [pltpu.VMEM((CE, D_MSG), jnp.float32) for _ in range(2)]
            + [pltpu.SemaphoreType.DMA for _ in range(12)]
        ),
    )
    return f(hw, ew, src1, dst1)


def _tail_body(agg_ref, h_ref, batch_ref, wupd_ref, wemb_ref,
               w0_ref, b0_ref, w1_ref, b1_ref, w2_ref, b2_ref, w3_ref, b3_ref,
               y_ref, acc_ref):
    i = pl.program_id(0)
    nblk = pl.num_programs(0)
    agg = (agg_ref[0] + agg_ref[1])[:, 0:D_MSG]
    h2 = h_ref[...] + jax.nn.gelu(
        jnp.dot(agg, wupd_ref[...], preferred_element_type=jnp.float32))
    g = jnp.dot(h2, wemb_ref[...], preferred_element_type=jnp.float32)
    brow = batch_ref[0]
    oht = (brow == lax.broadcasted_iota(jnp.int32, (B, brow.shape[1]), 0))
    contrib = jnp.dot(oht.astype(jnp.float32), g,
                      preferred_element_type=jnp.float32)

    @pl.when(i == 0)
    def _():
        acc_ref[...] = jnp.zeros_like(acc_ref)

    acc_ref[...] += contrib

    @pl.when(i == nblk - 1)
    def _():
        y = acc_ref[...]
        for w_ref, b_ref in ((w0_ref, b0_ref), (w1_ref, b1_ref),
                             (w2_ref, b2_ref), (w3_ref, b3_ref)):
            y = jax.nn.gelu(
                jnp.dot(y, w_ref[...], preferred_element_type=jnp.float32)
                + b_ref[...])
        y_ref[...] = y


def _tail(agg2, h, batch3, w_upd, w_emb, w0, b0, w1, b1, w2, b2, w3, b3):
    blk = 1000
    grid = N // blk
    fcspec = pl.BlockSpec((FC_DEPTH, FC_DEPTH), lambda i: (0, 0))
    bspec = pl.BlockSpec((1, FC_DEPTH), lambda i: (0, 0))
    return pl.pallas_call(
        _tail_body,
        grid=(grid,),
        in_specs=[
            pl.BlockSpec((NC, blk, 2 * D_MSG), lambda i: (0, i, 0)),
            pl.BlockSpec((blk, D_NODE), lambda i: (i, 0)),
            pl.BlockSpec((1, 1, blk), lambda i: (i, 0, 0)),
            pl.BlockSpec((D_MSG, D_NODE), lambda i: (0, 0)),
            pl.BlockSpec((D_NODE, D_EMB), lambda i: (0, 0)),
            pl.BlockSpec((D_EMB, FC_DEPTH), lambda i: (0, 0)),
            bspec, fcspec, bspec, fcspec, bspec, fcspec, bspec,
        ],
        out_specs=pl.BlockSpec((B, FC_DEPTH), lambda i: (0, 0)),
        out_shape=jax.ShapeDtypeStruct((B, FC_DEPTH), jnp.float32),
        scratch_shapes=[pltpu.VMEM((B, FC_DEPTH), jnp.float32)],
    )(agg2, h, batch3, w_upd, w_emb, w0, b0, w1, b1, w2, b2, w3, b3)


def kernel(x, pos, batch, edge_index, W_in, b_in, W_rbf, W_msg, W_upd, W_emb,
           W_fc0, b_fc0, W_fc1, b_fc1, W_fc2, b_fc2, W_fc3, b_fc3):
    src = edge_index[0].astype(jnp.int32)
    dst = edge_index[1].astype(jnp.int32)
    padz = jnp.zeros((EPAD - E,), jnp.int32)
    src1 = jnp.concatenate([src, padz])
    dst1 = jnp.concatenate([dst, padz])
    srcp = src1.reshape(EPAD // CB, CB)
    dstp = dst1.reshape(EPAD // CB, CB)
    px = pos[:, 0]
    py = pos[:, 1]
    pz = pos[:, 2]

    h, hw = _node_embed(x, W_in, b_in.reshape(1, D_NODE), W_msg)
    d2 = _dist2_sc(px, py, pz, srcp, dstp)
    ew = _edge_w(d2.reshape(EPAD // 128, 128), W_rbf)
    agg2 = _scatter_sc(hw, ew, src1, dst1).reshape(NC, N, 2 * D_MSG)
    batch3 = batch.astype(jnp.int32).reshape(N // 1000, 1, 1000)
    y = _tail(agg2, h, batch3, W_upd, W_emb,
              W_fc0, b_fc0.reshape(1, FC_DEPTH), W_fc1, b_fc1.reshape(1, FC_DEPTH),
              W_fc2, b_fc2.reshape(1, FC_DEPTH), W_fc3, b_fc3.reshape(1, FC_DEPTH))
    return y


# speedup vs baseline: 5.4736x; 1.0464x over previous
import functools

import jax
import jax.numpy as jnp
from jax import lax
from jax.experimental import pallas as pl
from jax.experimental.pallas import tpu as pltpu
from jax.experimental.pallas import tpu_sc as plsc

N = 10000
E = 320000
B = 100
D_IN = 128
D_NODE = 128
D_MSG = 64
D_EMB = 256
N_RADIAL = 32
CUTOFF = 6.0
FC_DEPTH = 256

NC = 2
NS = 16
NW = NC * NS
EPAD = 327680
EPT = EPAD // NW
CB = 128
NCB = EPT // CB
CE = 64
NCH = EPT // CE
NCH0 = 224
NCH1 = 2 * NCH - NCH0
E0TOT = NS * NCH0 * CE
RPT = N // NS

def _sc_mesh():
    return plsc.VectorSubcoreMesh(core_axis_name="c", subcore_axis_name="s")


_sc_params = pltpu.CompilerParams(needs_layout_passes=False)


def _node_embed_body(x_ref, win_ref, bin_ref, wmsg_ref, h_ref, hw_ref):
    h = jax.nn.gelu(
        jnp.dot(x_ref[...], win_ref[...], preferred_element_type=jnp.float32)
        + bin_ref[...]
    )
    h_ref[...] = h
    hw_ref[:, 0:D_MSG] = jnp.dot(h, wmsg_ref[...],
                                 preferred_element_type=jnp.float32)
    hw_ref[:, D_MSG:] = jnp.zeros_like(hw_ref[:, D_MSG:])


def _node_embed(x, w_in, b_in, w_msg):
    blk = 1000
    grid = N // blk
    return pl.pallas_call(
        _node_embed_body,
        grid=(grid,),
        in_specs=[
            pl.BlockSpec((blk, D_IN), lambda i: (i, 0)),
            pl.BlockSpec((D_IN, D_NODE), lambda i: (0, 0)),
            pl.BlockSpec((1, D_NODE), lambda i: (0, 0)),
            pl.BlockSpec((D_NODE, D_MSG), lambda i: (0, 0)),
        ],
        out_specs=[
            pl.BlockSpec((blk, D_NODE), lambda i: (i, 0)),
            pl.BlockSpec((blk, 2 * D_MSG), lambda i: (i, 0)),
        ],
        out_shape=[
            jax.ShapeDtypeStruct((N, D_NODE), jnp.float32),
            jax.ShapeDtypeStruct((N, 2 * D_MSG), jnp.float32),
        ],
    )(x, w_in, b_in, w_msg)


def _dist2_body(px_hbm, py_hbm, pz_hbm, src_hbm, dst_hbm, out_hbm,
                px_v, py_v, pz_v, src_v, dst_v, out_v):
    cid = lax.axis_index("c")
    sid = lax.axis_index("s")
    wid = sid * NC + cid
    base = wid * EPT
    pltpu.sync_copy(px_hbm, px_v)
    pltpu.sync_copy(py_hbm, py_v)
    pltpu.sync_copy(pz_hbm, pz_v)
    pltpu.sync_copy(src_hbm.at[pl.ds(wid * NCB, NCB)], src_v)
    pltpu.sync_copy(dst_hbm.at[pl.ds(wid * NCB, NCB)], dst_v)

    def body(r, carry):
        for cc in range(CB // 16):
            o = r * CB + cc * 16
            si = src_v[r, pl.ds(cc * 16, 16)]
            di = dst_v[r, pl.ds(cc * 16, 16)]
            dx = plsc.load_gather(px_v, [di]) - plsc.load_gather(px_v, [si])
            dy = plsc.load_gather(py_v, [di]) - plsc.load_gather(py_v, [si])
            dz = plsc.load_gather(pz_v, [di]) - plsc.load_gather(pz_v, [si])
            out_v[pl.ds(o, 16)] = dx * dx + dy * dy + dz * dz + 1e-8
        return carry

    lax.fori_loop(0, NCB, body, 0)
    pltpu.sync_copy(out_v, out_hbm.at[pl.ds(base, EPT)])


def _dist2_sc(px, py, pz, srcp, dstp):
    f = pl.kernel(
        _dist2_body,
        mesh=_sc_mesh(),
        compiler_params=_sc_params,
        out_type=jax.ShapeDtypeStruct((EPAD,), jnp.float32),
        scratch_types=[
            pltpu.VMEM((N,), jnp.float32),
            pltpu.VMEM((N,), jnp.float32),
            pltpu.VMEM((N,), jnp.float32),
            pltpu.VMEM((NCB, CB), jnp.int32),
            pltpu.VMEM((NCB, CB), jnp.int32),
            pltpu.VMEM((EPT,), jnp.float32),
        ],
    )
    return f(px, py, pz, srcp, dstp)


def _edge_w_body(d2_ref, s_ref, w4_ref, ew_ref):
    i = pl.program_id(0)
    rows = d2_ref.shape[0]
    dw = jnp.sqrt(d2_ref[...])
    gid = ((i * rows + lax.broadcasted_iota(jnp.int32, (rows, 128), 0)) * 128
           + lax.broadcasted_iota(jnp.int32, (rows, 128), 1))
    maskw = (gid < E).astype(jnp.float32)
    envw = (0.5 * (jnp.cos(jnp.pi * jnp.clip(dw * (1.0 / CUTOFF), 0.0, 1.0))
                   + 1.0)) * maskw
    dt = dw.T
    et = envw.T
    step = CUTOFF / (N_RADIAL - 1)
    c4 = (lax.broadcasted_iota(jnp.int32, (1, 128), 1) % N_RADIAL
          ).astype(jnp.float32) * step
    width = CUTOFF / N_RADIAL
    gamma = 1.0 / (2.0 * width * width)
    s_mat = s_ref[...]
    for g in range(rows // 4):
        d4 = jnp.dot(dt[:, 4 * g:4 * g + 4], s_mat,
                     preferred_element_type=jnp.float32)
        e4 = jnp.dot(et[:, 4 * g:4 * g + 4], s_mat,
                     preferred_element_type=jnp.float32)
        r4 = jnp.exp(-((d4 - c4) ** 2) * gamma) * e4
        ew4 = jnp.dot(r4, w4_ref[...], preferred_element_type=jnp.float32)
        for b in range(4):
            ew_ref[pl.ds((4 * g + b) * 128, 128), :] = (
                ew4[:, b * D_MSG:(b + 1) * D_MSG])


def _edge_w(d2_2d, w_rbf):
    rows = 64
    grid = (EPAD // 128) // rows
    s_mat = jnp.repeat(jnp.eye(4, dtype=jnp.float32), N_RADIAL, axis=1)
    w4 = jnp.kron(jnp.eye(4, dtype=jnp.float32), w_rbf)
    return pl.pallas_call(
        _edge_w_body,
        grid=(grid,),
        in_specs=[
            pl.BlockSpec((rows, 128), lambda i: (i, 0)),
            pl.BlockSpec((4, 128), lambda i: (0, 0)),
            pl.BlockSpec((4 * N_RADIAL, 4 * D_MSG), lambda i: (0, 0)),
        ],
        out_specs=pl.BlockSpec((rows * 128, D_MSG), lambda i: (i, 0)),
        out_shape=jax.ShapeDtypeStruct((EPAD, D_MSG), jnp.float32),
    )(d2_2d, s_mat, w4)


def _scatter_body(hw_hbm, ew_hbm, src_hbm, dst_hbm, out_hbm,
                  agg_sh,
                  is0, is1, is2, is3, is4, is5, is6, is7,
                  id0, id1, id2, id3, id4, id5, id6, id7,
                  rows0, rows1, ew0, ew1,
                  g0, g1, s0, s1,
                  i0, i1, i2, i3, i4, i5, i6, i7):
    cid = lax.axis_index("c")
    sid = lax.axis_index("s")
    wid = sid * NC + cid
    base = wid * EPT

    rows = (rows0, rows1)
    ews = (ew0, ew1)
    gsem = (g0, g1)
    ssem = (s0, s1)
    isrc = (is0, is1, is2, is3, is4, is5, is6, is7)
    idst = (id0, id1, id2, id3, id4, id5, id6, id7)
    isem = (i0, i1, i2, i3, i4, i5, i6, i7)

    def zrow(r, carry):
        for cc in range(2 * D_MSG // 16):
            rows0[r, pl.ds(cc * 16, 16)] = jnp.zeros((16,), jnp.float32)
        return carry

    lax.fori_loop(0, CE, zrow, 0)
    nz = RPT // CE
    for k in range(nz):
        pltpu.sync_copy(rows0, agg_sh.at[pl.ds(sid * RPT + k * CE, CE)])
    pltpu.sync_copy(rows0.at[pl.ds(0, RPT - nz * CE)],
                    agg_sh.at[pl.ds(sid * RPT + nz * CE, RPT - nz * CE)])
    plsc.subcore_barrier()

    def run_pipeline(nch, tbase):
        def iload(c, s8):
            eb = tbase + c * CE
            pltpu.async_copy(src_hbm.at[pl.ds(eb, CE)], isrc[s8], isem[s8])
            pltpu.async_copy(dst_hbm.at[pl.ds(eb, CE)], idst[s8], isem[s8])

        def iwait(c, s8):
            eb = tbase + c * CE
            pltpu.make_async_copy(src_hbm.at[pl.ds(eb, CE)], isrc[s8],
                                  isem[s8]).wait()
            pltpu.make_async_copy(dst_hbm.at[pl.ds(eb, CE)], idst[s8],
                                  isem[s8]).wait()

        def start(c, s2, s8):
            eb = tbase + c * CE
            pltpu.async_copy(hw_hbm.at[isrc[s8]], rows[s2], gsem[s2])
            pltpu.async_copy(ew_hbm.at[pl.ds(eb, CE)], ews[s2], gsem[s2])

        def swait(s2, s8):
            pltpu.make_async_copy(rows[s2], agg_sh.at[idst[s8]],
                                  ssem[s2]).wait()

        def finish(c, s2, s8):
            pltpu.make_async_copy(hw_hbm.at[isrc[s8]], rows[s2],
                                  gsem[s2]).wait()
            pltpu.make_async_copy(ew_hbm.at[pl.ds(tbase + c * CE, CE)],
                                  ews[s2], gsem[s2]).wait()

            def mrow(r, carry):
                for u in range(4):
                    rr = r * 4 + u
                    for cc in range(D_MSG // 16):
                        s = pl.ds(cc * 16, 16)
                        rows[s2][rr, s] = rows[s2][rr, s] * ews[s2][rr, s]
                return carry

            lax.fori_loop(0, CE // 4, mrow, 0)
            pltpu.async_copy(rows[s2], agg_sh.at[idst[s8]], ssem[s2],
                             add=True)

        R = 8
        iload(0, 0)
        iload(1, 1)
        iwait(0, 0)
        start(0, 0, 0)
        iload(2, 2)
        for c in range(1, R):
            iwait(c, c % 8)
            if c >= 2:
                swait((c - 2) % 2, (c - 2) % 8)
            start(c, c % 2, c % 8)
            iload(c + 2, (c + 2) % 8)
            finish(c - 1, (c - 1) % 2, (c - 1) % 8)

        def group(gi, carry):
            c0 = R + gi * R
            for b in range(R):
                c = c0 + b
                iwait(c, b % 8)
                swait(b % 2, (b - 2) % 8)
                start(c, b % 2, b % 8)
                iload(c + 2, (b + 2) % 8)
                finish(c - 1, (b - 1) % 2, (b - 1) % 8)
            return carry

        lax.fori_loop(0, (nch - 2 * R) // R, group, 0)

        for c in range(nch - R, nch):
            iwait(c, c % 8)
            swait((c - 2) % 2, (c - 2) % 8)
            start(c, c % 2, c % 8)
            if c + 2 < nch:
                iload(c + 2, (c + 2) % 8)
            finish(c - 1, (c - 1) % 2, (c - 1) % 8)
        finish(nch - 1, (nch - 1) % 2, (nch - 1) % 8)
        swait((nch - 2) % 2, (nch - 2) % 8)
        swait((nch - 1) % 2, (nch - 1) % 8)

    @pl.when(cid == 0)
    def _():
        run_pipeline(NCH0, sid * (NCH0 * CE))

    @pl.when(cid == 1)
    def _():
        run_pipeline(NCH1, E0TOT + sid * (NCH1 * CE))

    plsc.subcore_barrier()
    pltpu.sync_copy(agg_sh.at[pl.ds(sid * RPT, RPT)], out_hbm.at[cid, sid])


def _scatter_sc(hw, ew, src1, dst1):
    f = pl.kernel(
        _scatter_body,
        mesh=_sc_mesh(),
        compiler_params=_sc_params,
        out_type=jax.ShapeDtypeStruct((NC, NS, RPT, 2 * D_MSG), jnp.float32),
        scratch_types=(
            [pltpu.VMEM_SHARED((N, 2 * D_MSG), jnp.float32)]
            + [pltpu.VMEM((CE,), jnp.int32) for _ in range(16)]
            + [pltpu.VMEM((CE, 2 * D_MSG), jnp.float32) for _ in range(2)]
            + [pltpu.VMEM((CE, D_MSG), jnp.float32) for _ in range(2)]
            + [pltpu.SemaphoreType.DMA for _ in range(12)]
        ),
    )
    return f(hw, ew, src1, dst1)


def _tail_body(agg_ref, h_ref, batch_ref, wupd_ref, wemb_ref,
               w0_ref, b0_ref, w1_ref, b1_ref, w2_ref, b2_ref, w3_ref, b3_ref,
               y_ref, acc_ref):
    i = pl.program_id(0)
    nblk = pl.num_programs(0)
    agg = (agg_ref[0] + agg_ref[1])[:, 0:D_MSG]
    h2 = h_ref[...] + jax.nn.gelu(
        jnp.dot(agg, wupd_ref[...], preferred_element_type=jnp.float32))
    g = jnp.dot(h2, wemb_ref[...], preferred_element_type=jnp.float32)
    brow = batch_ref[0]
    oht = (brow == lax.broadcasted_iota(jnp.int32, (B, brow.shape[1]), 0))
    contrib = jnp.dot(oht.astype(jnp.float32), g,
                      preferred_element_type=jnp.float32)

    @pl.when(i == 0)
    def _():
        acc_ref[...] = jnp.zeros_like(acc_ref)

    acc_ref[...] += contrib

    @pl.when(i == nblk - 1)
    def _():
        y = acc_ref[...]
        for w_ref, b_ref in ((w0_ref, b0_ref), (w1_ref, b1_ref),
                             (w2_ref, b2_ref), (w3_ref, b3_ref)):
            y = jax.nn.gelu(
                jnp.dot(y, w_ref[...], preferred_element_type=jnp.float32)
                + b_ref[...])
        y_ref[...] = y


def _tail(agg2, h, batch3, w_upd, w_emb, w0, b0, w1, b1, w2, b2, w3, b3):
    blk = 1000
    grid = N // blk
    fcspec = pl.BlockSpec((FC_DEPTH, FC_DEPTH), lambda i: (0, 0))
    bspec = pl.BlockSpec((1, FC_DEPTH), lambda i: (0, 0))
    return pl.pallas_call(
        _tail_body,
        grid=(grid,),
        in_specs=[
            pl.BlockSpec((NC, blk, 2 * D_MSG), lambda i: (0, i, 0)),
            pl.BlockSpec((blk, D_NODE), lambda i: (i, 0)),
            pl.BlockSpec((1, 1, blk), lambda i: (i, 0, 0)),
            pl.BlockSpec((D_MSG, D_NODE), lambda i: (0, 0)),
            pl.BlockSpec((D_NODE, D_EMB), lambda i: (0, 0)),
            pl.BlockSpec((D_EMB, FC_DEPTH), lambda i: (0, 0)),
            bspec, fcspec, bspec, fcspec, bspec, fcspec, bspec,
        ],
        out_specs=pl.BlockSpec((B, FC_DEPTH), lambda i: (0, 0)),
        out_shape=jax.ShapeDtypeStruct((B, FC_DEPTH), jnp.float32),
        scratch_shapes=[pltpu.VMEM((B, FC_DEPTH), jnp.float32)],
    )(agg2, h, batch3, w_upd, w_emb, w0, b0, w1, b1, w2, b2, w3, b3)


def kernel(x, pos, batch, edge_index, W_in, b_in, W_rbf, W_msg, W_upd, W_emb,
           W_fc0, b_fc0, W_fc1, b_fc1, W_fc2, b_fc2, W_fc3, b_fc3):
    src = edge_index[0].astype(jnp.int32)
    dst = edge_index[1].astype(jnp.int32)
    padz = jnp.zeros((EPAD - E,), jnp.int32)
    src1 = jnp.concatenate([src, padz])
    dst1 = jnp.concatenate([dst, padz])
    srcp = src1.reshape(EPAD // CB, CB)
    dstp = dst1.reshape(EPAD // CB, CB)
    px = pos[:, 0]
    py = pos[:, 1]
    pz = pos[:, 2]

    h, hw = _node_embed(x, W_in, b_in.reshape(1, D_NODE), W_msg)
    d2 = _dist2_sc(px, py, pz, srcp, dstp)
    ew = _edge_w(d2.reshape(EPAD // 128, 128), W_rbf)
    agg2 = _scatter_sc(hw, ew, src1, dst1).reshape(NC, N, 2 * D_MSG)
    batch3 = batch.astype(jnp.int32).reshape(N // 1000, 1, 1000)
    y = _tail(agg2, h, batch3, W_upd, W_emb,
              W_fc0, b_fc0.reshape(1, FC_DEPTH), W_fc1, b_fc1.reshape(1, FC_DEPTH),
              W_fc2, b_fc2.reshape(1, FC_DEPTH), W_fc3, b_fc3.reshape(1, FC_DEPTH))
    return y
